# baseline (speedup 1.0000x reference)
import functools

import jax
import jax.numpy as jnp
from jax import lax
from jax.experimental import pallas as pl
from jax.experimental.pallas import tpu as pltpu
from jax.experimental.pallas import tpu_sc as plsc

NUM_FIELDS = 26
FIELD_SIZE = 100000
BATCH = 16384
TABLE_ROWS = NUM_FIELDS * FIELD_SIZE
MAIN_LIM = 2599936
TAIL = TABLE_ROWS - MAIN_LIM

NC, NS, L = 2, 16, 16
NW = NC * NS
B_PER_W = BATCH // NW
N_CHUNK = B_PER_W // L
F_G = NUM_FIELDS // 2
E_G = F_G * B_PER_W
LAST = NUM_FIELDS - 1


def _sc_body(xt_hbm, w_hbm, b_hbm, out_hbm,
             idxa_v, idxb_v, valsa_v, valsb_v, out_v,
             bias_v, sema, semb, gsema, gsemb):
    wid = lax.axis_index("s") * NC + lax.axis_index("c")
    obase = wid * B_PER_W

    copies_a = [
        pltpu.async_copy(xt_hbm.at[f, pl.ds(obase, B_PER_W)],
                         idxa_v.at[pl.ds(f * B_PER_W, B_PER_W)], sema)
        for f in range(F_G)
    ]
    copies_b = [
        pltpu.async_copy(xt_hbm.at[F_G + f, pl.ds(obase, B_PER_W)],
                         idxb_v.at[pl.ds(f * B_PER_W, B_PER_W)], semb)
        for f in range(F_G)
    ]
    pltpu.sync_copy(b_hbm, bias_v)

    for c in copies_a:
        c.wait()

    def add_a(j, _):
        for fl in range(1, F_G):
            sl = pl.ds(fl * B_PER_W + j * L, L)
            idxa_v[sl] = idxa_v[sl] + fl * FIELD_SIZE
        return _
    lax.fori_loop(0, N_CHUNK, add_a, None)
    ga = pltpu.async_copy(w_hbm.at[idxa_v], valsa_v, gsema)

    for c in copies_b:
        c.wait()

    def add_b(j, _):
        for fl in range(F_G):
            sl = pl.ds(fl * B_PER_W + j * L, L)
            idxb_v[sl] = idxb_v[sl] + (F_G + fl) * FIELD_SIZE
        return _
    lax.fori_loop(0, N_CHUNK, add_b, None)
    gb = pltpu.async_copy(w_hbm.at[idxb_v], valsb_v, gsemb)

    bias16 = bias_v[...]
    ga.wait()

    def red_a(j, _):
        acc = bias16
        for fl in range(F_G):
            acc = acc + valsa_v[pl.ds(fl * B_PER_W + j * L, L)]
        out_v[pl.ds(j * L, L)] = acc
        return _
    lax.fori_loop(0, N_CHUNK, red_a, None)

    gb.wait()

    def red_b(j, _):
        acc = out_v[pl.ds(j * L, L)]
        for fl in range(F_G):
            acc = acc + valsb_v[pl.ds(fl * B_PER_W + j * L, L)]
        out_v[pl.ds(j * L, L)] = acc
        return _
    lax.fori_loop(0, N_CHUNK, red_b, None)

    pltpu.sync_copy(out_v, out_hbm.at[pl.ds(obase, B_PER_W)])


@jax.jit
def _features_linear(xt, w, bias16):
    mesh = plsc.VectorSubcoreMesh(core_axis_name="c", subcore_axis_name="s",
                                  num_cores=NC, num_subcores=NS)
    out = pl.kernel(
        _sc_body,
        out_type=jax.ShapeDtypeStruct((BATCH,), jnp.float32),
        mesh=mesh,
        scratch_types=[
            pltpu.VMEM((E_G,), jnp.int32),
            pltpu.VMEM((E_G,), jnp.int32),
            pltpu.VMEM((E_G,), jnp.float32),
            pltpu.VMEM((E_G,), jnp.float32),
            pltpu.VMEM((B_PER_W,), jnp.float32),
            pltpu.VMEM((L,), jnp.float32),
            pltpu.SemaphoreType.DMA,
            pltpu.SemaphoreType.DMA,
            pltpu.SemaphoreType.DMA,
            pltpu.SemaphoreType.DMA,
        ],
        compiler_params=pltpu.CompilerParams(
            needs_layout_passes=False,
            use_tc_tiling_on_sc=True,
        ),
    )(xt, w, bias16)
    return out


def kernel(x, fc_weight, bias):
    xt = x.T
    w = fc_weight.reshape(-1)
    bias16 = jnp.broadcast_to(bias, (L,))
    out = _features_linear(xt, w, bias16)
    return out.reshape(BATCH, 1)
